# baseline (speedup 1.0000x reference)
import functools

import jax
import jax.numpy as jnp
from jax import lax
from jax.experimental import pallas as pl
from jax.experimental.pallas import tpu as pltpu
from jax.experimental.pallas import tpu_sc as plsc

_G = 4


def _sc_mesh_info():
    info = plsc.get_sparse_core_info()
    return info.num_cores, info.num_subcores


@functools.lru_cache(maxsize=None)
def _make_detile(V, D):
    NC, NS = _sc_mesh_info()
    NW = NC * NS
    FULL = V // 128
    REM = V - FULL * 128
    LROWS = (V * D) // 128
    assert FULL % _G == 0
    NGRP = FULL // _G
    TRIPS = (NGRP + NW - 1) // NW

    mesh = plsc.VectorSubcoreMesh(core_axis_name="c", subcore_axis_name="s")

    @functools.partial(
        pl.kernel,
        mesh=mesh,
        out_type=jax.ShapeDtypeStruct((LROWS, 128), jnp.float32),
        scratch_types=[
            pltpu.VMEM((D, 128 * _G), jnp.float32),
            pltpu.VMEM((D, 128 * _G), jnp.float32),
            pltpu.VMEM((D * _G, 128), jnp.float32),
            pltpu.VMEM((D * _G, 128), jnp.float32),
            pltpu.VMEM((D, 64), jnp.float32),
            pltpu.SemaphoreType.DMA,
            pltpu.SemaphoreType.DMA,
            pltpu.SemaphoreType.DMA,
            pltpu.SemaphoreType.DMA,
        ],
        compiler_params=pltpu.CompilerParams(needs_layout_passes=False),
    )
    def detile_kernel(lut_t_hbm, l_hbm, in0, in1, out0, out1, in_rem,
                      isem0, isem1, osem0, osem1):
        wid = lax.axis_index("s") * NC + lax.axis_index("c")
        inb = (in0, in1)
        outb = (out0, out1)
        isem = (isem0, isem1)
        osem = (osem0, osem1)

        ii = lax.iota(jnp.int32, 16)
        zz = ii * 0
        cvecs = [ii + 16 * (v % 2) for v in range(8)]
        aoffs = [v // 2 for v in range(8)]

        def grp_of(t):
            return wid + NW * t

        def start_in(t, b):
            g = grp_of(t)
            return pltpu.async_copy(
                lut_t_hbm.at[:, pl.ds(g * 128 * _G, 128 * _G)], inb[b],
                isem[b])

        def transpose_grp(b):
            @plsc.parallel_loop(0, D, carry=zz, unroll=2)
            def row_body(r, r4):
                for k in range(_G):
                    for v in range(8):
                        vals = plsc.load_gather(
                            inb[b],
                            [cvecs[v], r4 + (128 * k + aoffs[v])])
                        outb[b][D * k + r, pl.ds(16 * v, 16)] = vals
                return r4 + 4

        def start_out(t, b):
            g = grp_of(t)
            return pltpu.async_copy(
                outb[b], l_hbm.at[pl.ds(g * D * _G, D * _G)], osem[b])

        for b in range(2):
            @pl.when(grp_of(b) < NGRP)
            def _():
                start_in(b, b)

        def pair_body(kk, carry):
            for b in range(2):
                t = 2 * kk + b

                @pl.when(grp_of(t) < NGRP)
                def _():
                    pltpu.make_async_copy(
                        lut_t_hbm.at[:, pl.ds(0, 128 * _G)], inb[b],
                        isem[b]).wait()

                    @pl.when(t >= 2)
                    def _():
                        pltpu.make_async_copy(
                            outb[b],
                            l_hbm.at[pl.ds(0, D * _G)], osem[b]).wait()

                    transpose_grp(b)
                    start_out(t, b)

                    @pl.when(grp_of(t + 2) < NGRP)
                    def _():
                        start_in(t + 2, b)
            return carry

        lax.fori_loop(0, (TRIPS + 1) // 2, pair_body, 0)

        for b in range(2):
            @pl.when(grp_of(b) < NGRP)
            def _():
                pltpu.make_async_copy(
                    outb[b], l_hbm.at[pl.ds(0, D * _G)], osem[b]).wait()

        if REM:
            rrows = (REM * D) // 128
            @pl.when(wid == NW - 1)
            def _():
                pltpu.sync_copy(lut_t_hbm.at[:, pl.ds(FULL * 128, REM)],
                                in_rem)

                def row_body(r, r4):
                    for v in range(8):
                        vals = plsc.load_gather(
                            in_rem, [cvecs[v], r4 + aoffs[v]])
                        outb[0][r, pl.ds(16 * v, 16)] = vals
                    return r4 + 4
                lax.fori_loop(0, rrows, row_body, zz)
                pltpu.sync_copy(outb[0].at[pl.ds(0, rrows)],
                                l_hbm.at[pl.ds(FULL * D, rrows)])

    return detile_kernel


@functools.lru_cache(maxsize=None)
def _make_gather(V, D, B0, S):
    NC, NS = _sc_mesh_info()
    NW = NC * NS
    NTB = D // 8
    NJ = B0 // 128
    NPAIR = S * NJ
    assert NPAIR % (NW * _G) == 0 and NJ % _G == 0
    PER_W = NPAIR // NW
    NG_W = PER_W // _G
    assert NG_W % 2 == 0
    R_G = 128 * _G

    mesh = plsc.VectorSubcoreMesh(core_axis_name="c", subcore_axis_name="s")

    @functools.partial(
        pl.kernel,
        mesh=mesh,
        out_type=jax.ShapeDtypeStruct((S, NTB, NJ, 8, 128), jnp.float32),
        scratch_types=[
            pltpu.VMEM((R_G,), jnp.int32),
            pltpu.VMEM((R_G,), jnp.int32),
            pltpu.VMEM((R_G, D), jnp.float32),
            pltpu.VMEM((R_G, D), jnp.float32),
            pltpu.VMEM((NTB, _G, 8, 128), jnp.float32),
            pltpu.VMEM((NTB, _G, 8, 128), jnp.float32),
            pltpu.SemaphoreType.DMA,
            pltpu.SemaphoreType.DMA,
            pltpu.SemaphoreType.DMA,
            pltpu.SemaphoreType.DMA,
        ],
        compiler_params=pltpu.CompilerParams(
            use_tc_tiling_on_sc=False, needs_layout_passes=False),
    )
    def gather_kernel(l_hbm, tok_t_hbm, out_hbm, idx0, idx1,
                      rows0, rows1, tile0, tile1,
                      gsem0, gsem1, wsem0, wsem1):
        wid = lax.axis_index("s") * NC + lax.axis_index("c")
        idxb = (idx0, idx1)
        rowsb = (rows0, rows1)
        tileb = (tile0, tile1)
        gsem = (gsem0, gsem1)
        wsem = (wsem0, wsem1)

        ii = lax.iota(jnp.int32, 16)
        zz = ii * 0
        livecs = [ii + 16 * v for v in range(8)]

        def sj_of(t):
            p = wid * PER_W + t * _G
            return p // NJ, p % NJ

        def load_idx(t, b):
            s, j = sj_of(t)
            pltpu.sync_copy(tok_t_hbm.at[s, pl.ds(j * 128, R_G)], idxb[b])

        def start_gather(b):
            pltpu.async_copy(l_hbm.at[idxb[b]], rowsb[b], gsem[b])

        def transpose_grp(b):
            @plsc.parallel_loop(0, D, carry=zz, unroll=2)
            def row_body(i, ivec):
                tb = i // 8
                q = i - tb * 8
                for k in range(_G):
                    for v in range(8):
                        vals = plsc.load_gather(
                            rowsb[b], [livecs[v] + 128 * k, ivec])
                        tileb[b][tb, k, q, pl.ds(16 * v, 16)] = vals
                return ivec + 1

        def start_write(t, b):
            s, j = sj_of(t)
            for tb in range(NTB):
                pltpu.async_copy(
                    tileb[b].at[tb], out_hbm.at[s, tb, pl.ds(j, _G)],
                    wsem[b])

        def wait_gather(b):
            pltpu.make_async_copy(l_hbm.at[idxb[b]], rowsb[b], gsem[b]).wait()

        def wait_write(b):
            for tb in range(NTB):
                pltpu.make_async_copy(
                    tileb[b].at[tb], out_hbm.at[0, tb, pl.ds(0, _G)],
                    wsem[b]).wait()

        for b in range(2):
            load_idx(b, b)
            start_gather(b)

        def pair_body(kk, carry):
            for b in range(2):
                t = 2 * kk + b

                @pl.when(t >= 2)
                def _():
                    wait_write(b)

                wait_gather(b)
                transpose_grp(b)
                start_write(t, b)

                @pl.when(t + 2 < NG_W)
                def _():
                    load_idx(t + 2, b)
                    start_gather(b)
            return carry

        lax.fori_loop(0, NG_W // 2, pair_body, 0)

        for b in range(2):
            wait_write(b)

    return gather_kernel


def kernel(token_ids, embed_lut):
    B0, S = token_ids.shape
    V, D = embed_lut.shape
    lut_t = jnp.transpose(embed_lut)
    l_tab = _make_detile(V, D)(lut_t)
    l_lin = l_tab.reshape(V, D)
    tok_t = jnp.transpose(token_ids)
    p5 = _make_gather(V, D, B0, S)(l_lin, tok_t)
    out = jnp.transpose(p5, (2, 4, 0, 1, 3)).reshape(B0, S, D)
    return out


# speedup vs baseline: 1.1491x; 1.1491x over previous
import functools

import jax
import jax.numpy as jnp
from jax import lax
from jax.experimental import pallas as pl
from jax.experimental.pallas import tpu as pltpu
from jax.experimental.pallas import tpu_sc as plsc

_G = 4


def _sc_mesh_info():
    info = plsc.get_sparse_core_info()
    return info.num_cores, info.num_subcores


@functools.lru_cache(maxsize=None)
def _make_detile(V, D):
    NC, NS = _sc_mesh_info()
    NW = NC * NS
    FULL = V // 128
    REM = V - FULL * 128
    LROWS = (V * D) // 128
    assert FULL % _G == 0
    NGRP = FULL // _G
    TRIPS = (NGRP + NW - 1) // NW

    mesh = plsc.VectorSubcoreMesh(core_axis_name="c", subcore_axis_name="s")

    @functools.partial(
        pl.kernel,
        mesh=mesh,
        out_type=jax.ShapeDtypeStruct((LROWS, 128), jnp.float32),
        scratch_types=[
            pltpu.VMEM((D, 128 * _G), jnp.float32),
            pltpu.VMEM((D, 128 * _G), jnp.float32),
            pltpu.VMEM((D * _G, 128), jnp.float32),
            pltpu.VMEM((D * _G, 128), jnp.float32),
            pltpu.VMEM((D, 64), jnp.float32),
            pltpu.SemaphoreType.DMA,
            pltpu.SemaphoreType.DMA,
            pltpu.SemaphoreType.DMA,
            pltpu.SemaphoreType.DMA,
        ],
        compiler_params=pltpu.CompilerParams(needs_layout_passes=False),
    )
    def detile_kernel(lut_t_hbm, l_hbm, in0, in1, out0, out1, in_rem,
                      isem0, isem1, osem0, osem1):
        wid = lax.axis_index("s") * NC + lax.axis_index("c")
        inb = (in0, in1)
        outb = (out0, out1)
        isem = (isem0, isem1)
        osem = (osem0, osem1)

        ii = lax.iota(jnp.int32, 16)
        zz = ii * 0
        cvecs = [ii + 16 * (v % 2) for v in range(8)]
        aoffs = [v // 2 for v in range(8)]

        def grp_of(t):
            return wid + NW * t

        def start_in(t, b):
            g = grp_of(t)
            return pltpu.async_copy(
                lut_t_hbm.at[:, pl.ds(g * 128 * _G, 128 * _G)], inb[b],
                isem[b])

        def transpose_grp(b):
            @plsc.parallel_loop(0, D, carry=zz, unroll=4)
            def row_body(r, r4):
                for k in range(_G):
                    for v in range(8):
                        vals = plsc.load_gather(
                            inb[b],
                            [cvecs[v], r4 + (128 * k + aoffs[v])])
                        outb[b][D * k + r, pl.ds(16 * v, 16)] = vals
                return r4 + 4

        def start_out(t, b):
            g = grp_of(t)
            return pltpu.async_copy(
                outb[b], l_hbm.at[pl.ds(g * D * _G, D * _G)], osem[b])

        for b in range(2):
            @pl.when(grp_of(b) < NGRP)
            def _():
                start_in(b, b)

        def pair_body(kk, carry):
            for b in range(2):
                t = 2 * kk + b

                @pl.when(grp_of(t) < NGRP)
                def _():
                    pltpu.make_async_copy(
                        lut_t_hbm.at[:, pl.ds(0, 128 * _G)], inb[b],
                        isem[b]).wait()

                    @pl.when(t >= 2)
                    def _():
                        pltpu.make_async_copy(
                            outb[b],
                            l_hbm.at[pl.ds(0, D * _G)], osem[b]).wait()

                    transpose_grp(b)
                    start_out(t, b)

                    @pl.when(grp_of(t + 2) < NGRP)
                    def _():
                        start_in(t + 2, b)
            return carry

        lax.fori_loop(0, (TRIPS + 1) // 2, pair_body, 0)

        for b in range(2):
            @pl.when(grp_of(b) < NGRP)
            def _():
                pltpu.make_async_copy(
                    outb[b], l_hbm.at[pl.ds(0, D * _G)], osem[b]).wait()

        if REM:
            rrows = (REM * D) // 128
            @pl.when(wid == NW - 1)
            def _():
                pltpu.sync_copy(lut_t_hbm.at[:, pl.ds(FULL * 128, REM)],
                                in_rem)

                def row_body(r, r4):
                    for v in range(8):
                        vals = plsc.load_gather(
                            in_rem, [cvecs[v], r4 + aoffs[v]])
                        outb[0][r, pl.ds(16 * v, 16)] = vals
                    return r4 + 4
                lax.fori_loop(0, rrows, row_body, zz)
                pltpu.sync_copy(outb[0].at[pl.ds(0, rrows)],
                                l_hbm.at[pl.ds(FULL * D, rrows)])

    return detile_kernel


@functools.lru_cache(maxsize=None)
def _make_gather(V, D, B0, S):
    NC, NS = _sc_mesh_info()
    NW = NC * NS
    NTB = D // 8
    NJ = B0 // 128
    NPAIR = S * NJ
    assert NPAIR % (NW * _G) == 0 and NJ % _G == 0
    PER_W = NPAIR // NW
    NG_W = PER_W // _G
    assert NG_W % 2 == 0
    R_G = 128 * _G

    mesh = plsc.VectorSubcoreMesh(core_axis_name="c", subcore_axis_name="s")

    @functools.partial(
        pl.kernel,
        mesh=mesh,
        out_type=jax.ShapeDtypeStruct((S, NTB, NJ, 8, 128), jnp.float32),
        scratch_types=[
            pltpu.VMEM((R_G,), jnp.int32),
            pltpu.VMEM((R_G,), jnp.int32),
            pltpu.VMEM((R_G, D), jnp.float32),
            pltpu.VMEM((R_G, D), jnp.float32),
            pltpu.VMEM((NTB, _G, 8, 128), jnp.float32),
            pltpu.VMEM((NTB, _G, 8, 128), jnp.float32),
            pltpu.SemaphoreType.DMA,
            pltpu.SemaphoreType.DMA,
            pltpu.SemaphoreType.DMA,
            pltpu.SemaphoreType.DMA,
        ],
        compiler_params=pltpu.CompilerParams(
            use_tc_tiling_on_sc=False, needs_layout_passes=False),
    )
    def gather_kernel(l_hbm, tok_t_hbm, out_hbm, idx0, idx1,
                      rows0, rows1, tile0, tile1,
                      gsem0, gsem1, wsem0, wsem1):
        wid = lax.axis_index("s") * NC + lax.axis_index("c")
        idxb = (idx0, idx1)
        rowsb = (rows0, rows1)
        tileb = (tile0, tile1)
        gsem = (gsem0, gsem1)
        wsem = (wsem0, wsem1)

        ii = lax.iota(jnp.int32, 16)
        zz = ii * 0
        livecs = [ii + 16 * v for v in range(8)]

        def sj_of(t):
            p = wid * PER_W + t * _G
            return p // NJ, p % NJ

        def load_idx(t, b):
            s, j = sj_of(t)
            pltpu.sync_copy(tok_t_hbm.at[s, pl.ds(j * 128, R_G)], idxb[b])

        def start_gather(b):
            pltpu.async_copy(l_hbm.at[idxb[b]], rowsb[b], gsem[b])

        def transpose_grp(b):
            @plsc.parallel_loop(0, D, carry=zz, unroll=4)
            def row_body(i, ivec):
                tb = i // 8
                q = i - tb * 8
                for k in range(_G):
                    for v in range(8):
                        vals = plsc.load_gather(
                            rowsb[b], [livecs[v] + 128 * k, ivec])
                        tileb[b][tb, k, q, pl.ds(16 * v, 16)] = vals
                return ivec + 1

        def start_write(t, b):
            s, j = sj_of(t)
            for tb in range(NTB):
                pltpu.async_copy(
                    tileb[b].at[tb], out_hbm.at[s, tb, pl.ds(j, _G)],
                    wsem[b])

        def wait_gather(b):
            pltpu.make_async_copy(l_hbm.at[idxb[b]], rowsb[b], gsem[b]).wait()

        def wait_write(b):
            for tb in range(NTB):
                pltpu.make_async_copy(
                    tileb[b].at[tb], out_hbm.at[0, tb, pl.ds(0, _G)],
                    wsem[b]).wait()

        for b in range(2):
            load_idx(b, b)
            start_gather(b)

        def pair_body(kk, carry):
            for b in range(2):
                t = 2 * kk + b

                @pl.when(t >= 2)
                def _():
                    wait_write(b)

                wait_gather(b)
                transpose_grp(b)
                start_write(t, b)

                @pl.when(t + 2 < NG_W)
                def _():
                    load_idx(t + 2, b)
                    start_gather(b)
            return carry

        lax.fori_loop(0, NG_W // 2, pair_body, 0)

        for b in range(2):
            wait_write(b)

    return gather_kernel


def kernel(token_ids, embed_lut):
    B0, S = token_ids.shape
    V, D = embed_lut.shape
    lut_t = jnp.transpose(embed_lut)
    l_tab = _make_detile(V, D)(lut_t)
    l_lin = l_tab.reshape(V, D)
    tok_t = jnp.transpose(token_ids)
    p5 = _make_gather(V, D, B0, S)(l_lin, tok_t)
    out = jnp.transpose(p5, (2, 4, 0, 1, 3)).reshape(B0, S, D)
    return out
